# baseline (speedup 1.0000x reference)
import functools

import jax
import jax.numpy as jnp
from jax import lax
from jax.experimental import pallas as pl
from jax.experimental.pallas import tpu as pltpu
from jax.experimental.pallas import tpu_sc as plsc

NUM_USERS = 1000000
NUM_ITEMS = 1000000
EMB = 32
HID = 64
B = 16384

NC = 2
NS = 16
NW = NC * NS
IDX_CHUNK = 128
N_CHUNKS_TOTAL = B // IDX_CHUNK
CHUNKS_PER_W = N_CHUNKS_TOTAL // NW


def _gather_body(uid_hbm, iid_hbm, utab_hbm, itab_hbm, uout_hbm, iout_hbm,
                 uidx_v, iidx_v, urows_v, irows_v, usem, isem):
    wid = lax.axis_index("s") * NC + lax.axis_index("c")
    base = wid * CHUNKS_PER_W
    pltpu.sync_copy(uid_hbm.at[pl.ds(base, CHUNKS_PER_W)], uidx_v)
    pltpu.sync_copy(iid_hbm.at[pl.ds(base, CHUNKS_PER_W)], iidx_v)
    ucps = [pltpu.async_copy(utab_hbm.at[uidx_v.at[j]], urows_v.at[j], usem)
            for j in range(CHUNKS_PER_W)]
    icps = [pltpu.async_copy(itab_hbm.at[iidx_v.at[j]], irows_v.at[j], isem)
            for j in range(CHUNKS_PER_W)]
    for c in ucps:
        c.wait()
    for c in icps:
        c.wait()
    pltpu.sync_copy(urows_v, uout_hbm.at[pl.ds(base, CHUNKS_PER_W)])
    pltpu.sync_copy(irows_v, iout_hbm.at[pl.ds(base, CHUNKS_PER_W)])


@functools.cache
def _sc_gather_fn():
    return pl.kernel(
        _gather_body,
        out_type=[
            jax.ShapeDtypeStruct((N_CHUNKS_TOTAL, IDX_CHUNK, EMB), jnp.float32),
            jax.ShapeDtypeStruct((N_CHUNKS_TOTAL, IDX_CHUNK, EMB), jnp.float32),
        ],
        mesh=plsc.VectorSubcoreMesh(core_axis_name="c", subcore_axis_name="s"),
        scratch_types=[
            pltpu.VMEM((CHUNKS_PER_W, IDX_CHUNK), jnp.int32),
            pltpu.VMEM((CHUNKS_PER_W, IDX_CHUNK), jnp.int32),
            pltpu.VMEM((CHUNKS_PER_W, IDX_CHUNK, EMB), jnp.float32),
            pltpu.VMEM((CHUNKS_PER_W, IDX_CHUNK, EMB), jnp.float32),
            pltpu.SemaphoreType.DMA,
            pltpu.SemaphoreType.DMA,
        ],
        compiler_params=pltpu.CompilerParams(use_tc_tiling_on_sc=False),
    )


MLP_BLOCK = 2048


def _mlp_body(ue_ref, ie_ref, w1u_ref, w1i_ref, b1_ref, w2_ref, b2_ref,
              w3_ref, b3_ref, out_ref):
    h = (jnp.dot(ue_ref[...], w1u_ref[...], preferred_element_type=jnp.float32)
         + jnp.dot(ie_ref[...], w1i_ref[...], preferred_element_type=jnp.float32)
         + b1_ref[...])
    h = jnp.maximum(h, 0.0)
    h = jnp.dot(h, w2_ref[...], preferred_element_type=jnp.float32) + b2_ref[...]
    h = jnp.maximum(h, 0.0)
    out_ref[...] = jnp.sum(h * w3_ref[...], axis=1) + b3_ref[0]


def _mlp(user_emb, item_emb, w1u, w1i, b1, w2, b2, w3, b3):
    grid = B // MLP_BLOCK
    rep2 = lambda shape: pl.BlockSpec(shape, lambda i: (0, 0))
    return pl.pallas_call(
        _mlp_body,
        grid=(grid,),
        in_specs=[
            pl.BlockSpec((MLP_BLOCK, EMB), lambda i: (i, 0)),
            pl.BlockSpec((MLP_BLOCK, EMB), lambda i: (i, 0)),
            rep2((EMB, HID)),
            rep2((EMB, HID)),
            rep2((1, HID)),
            rep2((HID, HID // 2)),
            rep2((1, HID // 2)),
            rep2((1, HID // 2)),
            pl.BlockSpec(memory_space=pltpu.SMEM),
        ],
        out_specs=pl.BlockSpec((MLP_BLOCK,), lambda i: (i,)),
        out_shape=jax.ShapeDtypeStruct((B,), jnp.float32),
    )(user_emb, item_emb, w1u, w1i, b1, w2, b2, w3, b3)


def kernel(user_id, item_id, user_table, item_table, W1, b1, W2, b2, W3, b3):
    uid2 = user_id.reshape(N_CHUNKS_TOTAL, IDX_CHUNK)
    iid2 = item_id.reshape(N_CHUNKS_TOTAL, IDX_CHUNK)
    ue3, ie3 = _sc_gather_fn()(uid2, iid2, user_table, item_table)
    user_emb = ue3.reshape(B, EMB)
    item_emb = ie3.reshape(B, EMB)
    w1u = W1[:, :EMB].T
    w1i = W1[:, EMB:].T
    return _mlp(user_emb, item_emb, w1u, w1i, b1[None, :], W2.T, b2[None, :],
                W3, b3)


# speedup vs baseline: 3.9138x; 3.9138x over previous
import functools

import jax
import jax.numpy as jnp
from jax import lax
from jax.experimental import pallas as pl
from jax.experimental.pallas import tpu as pltpu
from jax.experimental.pallas import tpu_sc as plsc

NUM_USERS = 1000000
NUM_ITEMS = 1000000
EMB = 32
HID = 64
B = 16384

NC = 2
NS = 16
NW = NC * NS
BPW = B // NW
LANES = 128
RING = 8


GRP = 16


def _gather_body(uid_hbm, iid_hbm, utabT_hbm, itabT_hbm, uoutT_hbm, ioutT_hbm,
                 uidx_v, iidx_v, ubufs, ibufs, uout_v, iout_v, usems, isems):
    wid = lax.axis_index("s") * NC + lax.axis_index("c")
    base = wid * BPW
    pltpu.sync_copy(uid_hbm.at[pl.ds(base, BPW)], uidx_v)
    pltpu.sync_copy(iid_hbm.at[pl.ds(base, BPW)], iidx_v)

    rows0 = lax.iota(jnp.int32, 16)
    rows1 = rows0 + 16

    def fetch(ur, ir, slot):
        urt = pl.multiple_of((ur // LANES) * LANES, LANES)
        irt = pl.multiple_of((ir // LANES) * LANES, LANES)
        pltpu.async_copy(utabT_hbm.at[:, pl.ds(urt, LANES)], ubufs[slot],
                         usems[slot])
        pltpu.async_copy(itabT_hbm.at[:, pl.ds(irt, LANES)], ibufs[slot],
                         isems[slot])

    def wait(slot):
        pltpu.make_async_copy(utabT_hbm.at[:, pl.ds(0, LANES)], ubufs[slot],
                              usems[slot]).wait()
        pltpu.make_async_copy(itabT_hbm.at[:, pl.ds(0, LANES)], ibufs[slot],
                              isems[slot]).wait()

    def extract(i, ur, ir, slot):
        cols = jnp.full((16,), i, jnp.int32)
        url = jnp.full((16,), ur % LANES, jnp.int32)
        irl = jnp.full((16,), ir % LANES, jnp.int32)
        plsc.store_scatter(uout_v, [rows0, cols],
                           plsc.load_gather(ubufs[slot], [rows0, url]))
        plsc.store_scatter(uout_v, [rows1, cols],
                           plsc.load_gather(ubufs[slot], [rows1, url]))
        plsc.store_scatter(iout_v, [rows0, cols],
                           plsc.load_gather(ibufs[slot], [rows0, irl]))
        plsc.store_scatter(iout_v, [rows1, cols],
                           plsc.load_gather(ibufs[slot], [rows1, irl]))

    uv0 = uidx_v[pl.ds(0, GRP)]
    iv0 = iidx_v[pl.ds(0, GRP)]
    for p in range(RING):
        fetch(uv0[p], iv0[p], p)

    def body(g, carry):
        i0 = g * GRP
        uv = uidx_v[pl.ds(i0, GRP)]
        iv = iidx_v[pl.ds(i0, GRP)]
        nxt0 = jnp.minimum(i0 + GRP, BPW - GRP)
        uvn = uidx_v[pl.ds(nxt0, GRP)]
        ivn = iidx_v[pl.ds(nxt0, GRP)]
        for s in range(GRP):
            slot = s % RING
            wait(slot)
            extract(i0 + s, uv[s], iv[s], slot)
            if s < GRP - RING:
                fetch(uv[s + RING], iv[s + RING], slot)
            else:
                @pl.when(g < BPW // GRP - 1)
                def _():
                    fetch(uvn[s - (GRP - RING)], ivn[s - (GRP - RING)], slot)
        return carry

    lax.fori_loop(0, BPW // GRP, body, 0)
    pltpu.sync_copy(uout_v, uoutT_hbm.at[:, pl.ds(base, BPW)])
    pltpu.sync_copy(iout_v, ioutT_hbm.at[:, pl.ds(base, BPW)])


def _gather_entry(uid_hbm, iid_hbm, utabT_hbm, itabT_hbm, uoutT_hbm,
                  ioutT_hbm, *scr):
    uidx_v, iidx_v = scr[0], scr[1]
    ubufs = list(scr[2:2 + RING])
    ibufs = list(scr[2 + RING:2 + 2 * RING])
    uout_v, iout_v = scr[2 + 2 * RING], scr[3 + 2 * RING]
    usems = list(scr[4 + 2 * RING:4 + 3 * RING])
    isems = list(scr[4 + 3 * RING:4 + 4 * RING])
    _gather_body(uid_hbm, iid_hbm, utabT_hbm, itabT_hbm, uoutT_hbm, ioutT_hbm,
                 uidx_v, iidx_v, ubufs, ibufs, uout_v, iout_v, usems, isems)


@functools.cache
def _sc_gather_fn():
    buf = pltpu.VMEM((EMB, LANES), jnp.float32)
    return pl.kernel(
        _gather_entry,
        out_type=[
            jax.ShapeDtypeStruct((EMB, B), jnp.float32),
            jax.ShapeDtypeStruct((EMB, B), jnp.float32),
        ],
        mesh=plsc.VectorSubcoreMesh(core_axis_name="c", subcore_axis_name="s"),
        scratch_types=(
            [pltpu.VMEM((BPW,), jnp.int32)] * 2
            + [buf] * (2 * RING)
            + [pltpu.VMEM((EMB, BPW), jnp.float32)] * 2
            + [pltpu.SemaphoreType.DMA] * (2 * RING)
        ),
        compiler_params=pltpu.CompilerParams(use_tc_tiling_on_sc=True,
                                             needs_layout_passes=False),
    )


MLP_BLOCK = 2048


def _mlp_body(ue_ref, ie_ref, w1u_ref, w1i_ref, b1_ref, w2_ref, b2_ref,
              w3_ref, b3_ref, out_ref):
    h = (jnp.dot(w1u_ref[...], ue_ref[...], preferred_element_type=jnp.float32)
         + jnp.dot(w1i_ref[...], ie_ref[...], preferred_element_type=jnp.float32)
         + b1_ref[...])
    h = jnp.maximum(h, 0.0)
    h = jnp.dot(w2_ref[...], h, preferred_element_type=jnp.float32) + b2_ref[...]
    h = jnp.maximum(h, 0.0)
    out_ref[...] = jnp.sum(h * w3_ref[...], axis=0) + b3_ref[0]


def _mlp(ueT, ieT, w1u, w1i, b1c, w2, b2c, w3c, b3):
    grid = B // MLP_BLOCK
    rep2 = lambda shape: pl.BlockSpec(shape, lambda i: (0, 0))
    return pl.pallas_call(
        _mlp_body,
        grid=(grid,),
        in_specs=[
            pl.BlockSpec((EMB, MLP_BLOCK), lambda i: (0, i)),
            pl.BlockSpec((EMB, MLP_BLOCK), lambda i: (0, i)),
            rep2((HID, EMB)),
            rep2((HID, EMB)),
            rep2((HID, 1)),
            rep2((HID // 2, HID)),
            rep2((HID // 2, 1)),
            rep2((HID // 2, 1)),
            pl.BlockSpec(memory_space=pltpu.SMEM),
        ],
        out_specs=pl.BlockSpec((MLP_BLOCK,), lambda i: (i,)),
        out_shape=jax.ShapeDtypeStruct((B,), jnp.float32),
    )(ueT, ieT, w1u, w1i, b1c, w2, b2c, w3c, b3)


def kernel(user_id, item_id, user_table, item_table, W1, b1, W2, b2, W3, b3):
    ueT, ieT = _sc_gather_fn()(user_id, item_id, user_table.T, item_table.T)
    w1u = W1[:, :EMB]
    w1i = W1[:, EMB:]
    return _mlp(ueT, ieT, w1u, w1i, b1[:, None], W2, b2[:, None],
                W3[0][:, None], b3)
